# baseline (speedup 1.0000x reference)
import functools

import jax
import jax.numpy as jnp
from jax import lax
from jax.experimental import pallas as pl
from jax.experimental.pallas import tpu as pltpu
from jax.experimental.pallas import tpu_sc as plsc

_B = 16384
_D = 32
_R = 8
_NC = 2
_NS = 16
_NW = _NC * _NS
_BPW = _B // _NW
_L = 16
_CH = 64
_NCH = _BPW // _CH


def _sc_dot_kernel(user_hbm, item_hbm, ut_hbm, it_hbm, out_hbm,
                   uidx_v, iidx_v, usup_v, isup_v,
                   ubuf_v, ibuf_v, out_v, usem, isem):
    wid = lax.axis_index("s") * _NC + lax.axis_index("c")
    base = wid * _BPW

    pltpu.sync_copy(user_hbm.at[pl.ds(base, _BPW)], uidx_v)
    pltpu.sync_copy(item_hbm.at[pl.ds(base, _BPW)], iidx_v)

    for v in range(_BPW // _L):
        sl = pl.ds(v * _L, _L)
        usup_v[sl] = lax.shift_right_logical(uidx_v[sl], 3)
        isup_v[sl] = lax.shift_right_logical(iidx_v[sl], 3)

    lanes = lax.iota(jnp.int32, _L)

    def start(c):
        sl = pl.ds(c * _CH, _CH)
        slot = c % 2
        uc = pltpu.async_copy(ut_hbm.at[usup_v.at[sl]], ubuf_v.at[slot], usem)
        ic = pltpu.async_copy(it_hbm.at[isup_v.at[sl]], ibuf_v.at[slot], isem)
        return uc, ic

    cps = start(0)
    for c in range(_NCH):
        cps[0].wait()
        cps[1].wait()
        if c + 1 < _NCH:
            nxt = start(c + 1)
        slot = c % 2
        for g in range(_CH // _L):
            bsl = pl.ds(c * _CH + g * _L, _L)
            urem = jnp.bitwise_and(uidx_v[bsl], 7)
            irem = jnp.bitwise_and(iidx_v[bsl], 7)
            jloc = g * _L + lanes
            acc = jnp.zeros((_L,), jnp.float32)
            for k in range(_D):
                kv = jnp.full((_L,), k, jnp.int32)
                u = plsc.load_gather(ubuf_v.at[slot], [jloc, urem, kv])
                w = plsc.load_gather(ibuf_v.at[slot], [jloc, irem, kv])
                acc = acc + u * w
            out_v[bsl] = acc
        if c + 1 < _NCH:
            cps = nxt

    pltpu.sync_copy(out_v, out_hbm.at[pl.ds(base, _BPW)])


@jax.jit
def _run(user, item, user_table, item_table):
    ut3 = user_table.reshape(-1, _R, _D)
    it3 = item_table.reshape(-1, _R, _D)
    mesh = plsc.VectorSubcoreMesh(core_axis_name="c", subcore_axis_name="s")
    f = functools.partial(
        pl.kernel,
        mesh=mesh,
        out_type=jax.ShapeDtypeStruct((_B,), jnp.float32),
        scratch_types=[
            pltpu.VMEM((_BPW,), jnp.int32),
            pltpu.VMEM((_BPW,), jnp.int32),
            pltpu.VMEM((_BPW,), jnp.int32),
            pltpu.VMEM((_BPW,), jnp.int32),
            pltpu.VMEM((2, _CH, _R, _D), jnp.float32),
            pltpu.VMEM((2, _CH, _R, _D), jnp.float32),
            pltpu.VMEM((_BPW,), jnp.float32),
            pltpu.SemaphoreType.DMA,
            pltpu.SemaphoreType.DMA,
        ],
        compiler_params=pltpu.CompilerParams(
            needs_layout_passes=False, use_tc_tiling_on_sc=False),
    )(_sc_dot_kernel)
    return f(user, item, ut3, it3)


def kernel(user, item, user_table, item_table):
    return _run(user.astype(jnp.int32), item.astype(jnp.int32),
                user_table, item_table)


# speedup vs baseline: 2.3210x; 2.3210x over previous
import functools

import jax
import jax.numpy as jnp
from jax import lax
from jax.experimental import pallas as pl
from jax.experimental.pallas import tpu as pltpu
from jax.experimental.pallas import tpu_sc as plsc

_B = 16384
_D = 32
_R = 8
_NC = 2
_NS = 16
_NW = _NC * _NS
_BPW = _B // _NW
_L = 16
_NCHUNK = _BPW // _L


def _sc_dot_kernel(user_hbm, item_hbm, ut_hbm, it_hbm, out_hbm,
                   uidx_v, iidx_v, ubuf_v, ibuf_v, out_v,
                   usem0, usem1, isem0, isem1):
    wid = lax.axis_index("s") * _NC + lax.axis_index("c")
    base = wid * _BPW

    pltpu.sync_copy(user_hbm.at[pl.ds(base, _BPW)], uidx_v)
    pltpu.sync_copy(item_hbm.at[pl.ds(base, _BPW)], iidx_v)

    usems = (usem0, usem1)
    isems = (isem0, isem1)
    lanes = lax.iota(jnp.int32, _L)

    def fire(c, slot):
        sl = pl.ds(c * _L, _L)
        usup = lax.shift_right_logical(uidx_v[sl], 3)
        isup = lax.shift_right_logical(iidx_v[sl], 3)
        for k in range(_L):
            pltpu.async_copy(ut_hbm.at[pl.ds(usup[k], 1)],
                             ubuf_v.at[slot].at[pl.ds(k, 1)], usems[slot])
            pltpu.async_copy(it_hbm.at[pl.ds(isup[k], 1)],
                             ibuf_v.at[slot].at[pl.ds(k, 1)], isems[slot])

    def wait(slot):
        pltpu.make_async_copy(ut_hbm.at[pl.ds(0, _L)],
                              ubuf_v.at[slot], usems[slot]).wait()
        pltpu.make_async_copy(it_hbm.at[pl.ds(0, _L)],
                              ibuf_v.at[slot], isems[slot]).wait()

    def compute(c, slot):
        sl = pl.ds(c * _L, _L)
        urem = jnp.bitwise_and(uidx_v[sl], 7)
        irem = jnp.bitwise_and(iidx_v[sl], 7)
        acc = jnp.zeros((_L,), jnp.float32)
        for k in range(_D):
            kv = jnp.full((_L,), k, jnp.int32)
            u = plsc.load_gather(ubuf_v.at[slot], [lanes, urem, kv])
            w = plsc.load_gather(ibuf_v.at[slot], [lanes, irem, kv])
            acc = acc + u * w
        out_v[sl] = acc

    fire(0, 0)

    def body(i, carry):
        c0 = i * 2
        fire(c0 + 1, 1)
        wait(0)
        compute(c0, 0)
        fire(lax.rem(c0 + 2, _NCHUNK), 0)
        wait(1)
        compute(c0 + 1, 1)
        return carry

    lax.fori_loop(0, _NCHUNK // 2, body, 0)
    wait(0)

    pltpu.sync_copy(out_v, out_hbm.at[pl.ds(base, _BPW)])


@jax.jit
def _run(user, item, user_table, item_table):
    ut3 = user_table.reshape(-1, _R, _D)
    it3 = item_table.reshape(-1, _R, _D)
    mesh = plsc.VectorSubcoreMesh(core_axis_name="c", subcore_axis_name="s")
    f = functools.partial(
        pl.kernel,
        mesh=mesh,
        out_type=jax.ShapeDtypeStruct((_B,), jnp.float32),
        scratch_types=[
            pltpu.VMEM((_BPW,), jnp.int32),
            pltpu.VMEM((_BPW,), jnp.int32),
            pltpu.VMEM((2, _L, _R, _D), jnp.float32),
            pltpu.VMEM((2, _L, _R, _D), jnp.float32),
            pltpu.VMEM((_BPW,), jnp.float32),
            pltpu.SemaphoreType.DMA,
            pltpu.SemaphoreType.DMA,
            pltpu.SemaphoreType.DMA,
            pltpu.SemaphoreType.DMA,
        ],
        compiler_params=pltpu.CompilerParams(needs_layout_passes=False),
    )(_sc_dot_kernel)
    return f(user, item, ut3, it3)


def kernel(user, item, user_table, item_table):
    return _run(user.astype(jnp.int32), item.astype(jnp.int32),
                user_table, item_table)
